# baseline (speedup 1.0000x reference)
import functools

import jax
import jax.numpy as jnp
from jax import lax
from jax.experimental import pallas as pl
from jax.experimental.pallas import tpu as pltpu
from jax.experimental.pallas import tpu_sc as plsc

D = 1024
L = 16
NC = 2
NS = 16
NW = NC * NS
SCALE = 32.0

CHUNK = 32


def _sc_embed(ids3, weight, total_rows):
    nchunk = ids3.shape[1]
    mesh = plsc.VectorSubcoreMesh(core_axis_name="c", subcore_axis_name="s")

    @functools.partial(
        pl.kernel,
        mesh=mesh,
        out_type=jax.ShapeDtypeStruct((total_rows, D), jnp.float32),
        scratch_types=[
            pltpu.VMEM((nchunk, CHUNK), jnp.int32),
            pltpu.VMEM((CHUNK, D), jnp.float32),
            pltpu.SemaphoreType.DMA,
        ],
    )
    def k(ids_hbm, w_hbm, out_hbm, idx_v, buf, sem):
        wid = lax.axis_index("s") * NC + lax.axis_index("c")
        pltpu.sync_copy(ids_hbm.at[wid], idx_v)
        base = wid * (nchunk * CHUNK)

        def chunk_body(c, carry):
            pltpu.async_copy(w_hbm.at[idx_v.at[c]], buf, sem).wait()

            def row_body(i, rcarry):
                for j in range(D // L):
                    sl = pl.ds(j * L, L)
                    buf[i, sl] = buf[i, sl] * SCALE
                return rcarry

            lax.fori_loop(0, CHUNK, row_body, 0)
            pltpu.sync_copy(buf, out_hbm.at[pl.ds(base + c * CHUNK, CHUNK)])
            return carry

        lax.fori_loop(0, nchunk, chunk_body, 0)

    return k(ids3, weight)


def kernel(input_ids, weight):
    b, s = input_ids.shape
    total = b * s
    nchunk = total // (NW * CHUNK)
    ids3 = input_ids.astype(jnp.int32).reshape(NW, nchunk, CHUNK)
    out = _sc_embed(ids3, weight, total)
    return out.reshape(b, s, D)


# speedup vs baseline: 1.1711x; 1.1711x over previous
import functools

import jax
import jax.numpy as jnp
from jax import lax
from jax.experimental import pallas as pl
from jax.experimental.pallas import tpu as pltpu
from jax.experimental.pallas import tpu_sc as plsc

D = 1024
L = 16
NC = 2
NS = 16
NW = NC * NS
SCALE = 32.0

CHUNK = 16
NBUF = 2


def _sc_embed(ids3, weight, total_rows):
    nchunk = ids3.shape[1]
    assert nchunk % NBUF == 0 and nchunk >= 2 * NBUF
    mesh = plsc.VectorSubcoreMesh(core_axis_name="c", subcore_axis_name="s")

    @functools.partial(
        pl.kernel,
        mesh=mesh,
        out_type=jax.ShapeDtypeStruct((total_rows, D), jnp.float32),
        scratch_types=[
            pltpu.VMEM((nchunk, CHUNK), jnp.int32),
            pltpu.VMEM((NBUF, CHUNK, D), jnp.float32),
            pltpu.VMEM((NBUF, CHUNK, D), jnp.float32),
            pltpu.SemaphoreType.DMA,
            pltpu.SemaphoreType.DMA,
            pltpu.SemaphoreType.DMA,
            pltpu.SemaphoreType.DMA,
        ],
    )
    def k(ids_hbm, w_hbm, out_hbm, idx_v, gbuf, sbuf, gsem0, gsem1, ssem0, ssem1):
        gsems = [gsem0, gsem1]
        ssems = [ssem0, ssem1]
        wid = lax.axis_index("s") * NC + lax.axis_index("c")
        pltpu.sync_copy(ids_hbm.at[wid], idx_v)
        base = wid * (nchunk * CHUNK)

        def gather(c, b):
            return pltpu.make_async_copy(
                w_hbm.at[idx_v.at[c]], gbuf.at[b], gsems[b])

        def scatter(c, b):
            return pltpu.make_async_copy(
                sbuf.at[b], out_hbm.at[pl.ds(base + c * CHUNK, CHUNK)], ssems[b])

        for b in range(NBUF):
            gather(b, b).start()

        def ring_body(g, carry):
            for b in range(NBUF):
                c = g + b
                gather(c, b).wait()

                @pl.when(g > 0)
                def _():
                    scatter(c - NBUF, b).wait()

                def row_body(i, rcarry):
                    for j in range(D // L):
                        sl = pl.ds(j * L, L)
                        sbuf[b, i, sl] = gbuf[b, i, sl] * SCALE
                    return rcarry

                lax.fori_loop(0, CHUNK, row_body, 0)

                @pl.when(g < nchunk - NBUF)
                def _():
                    gather(c + NBUF, b).start()

                scatter(c, b).start()
            return carry

        lax.fori_loop(0, nchunk // NBUF, lambda t, cy: ring_body(t * NBUF, cy), 0)

        for b in range(NBUF):
            scatter(nchunk - NBUF + b, b).wait()

    return k(ids3, weight)


def kernel(input_ids, weight):
    b, s = input_ids.shape
    total = b * s
    nchunk = total // (NW * CHUNK)
    ids3 = input_ids.astype(jnp.int32).reshape(NW, nchunk, CHUNK)
    out = _sc_embed(ids3, weight, total)
    return out.reshape(b, s, D)
